# baseline (speedup 1.0000x reference)
import functools

import jax
import jax.numpy as jnp
import numpy as np
from jax import lax
from jax.experimental import pallas as pl
from jax.experimental.pallas import tpu as pltpu
from jax.experimental.pallas import tpu_sc as plsc

_N_ROWS = 100000
_SLICE = 16384
_D = 128

_NC = 2
_NS = 16
_NW = _NC * _NS
_B_PER_W = _SLICE // _NW
_CHUNK = 128
_N_CHUNKS = _B_PER_W // _CHUNK

_INDEX_CACHE = None


def _fixed_index() -> np.ndarray:
    global _INDEX_CACHE
    if _INDEX_CACHE is None:
        with jax.ensure_compile_time_eval():
            perm = jax.random.permutation(jax.random.key(1), _N_ROWS)
            _INDEX_CACHE = np.asarray(perm[:_SLICE])
    return _INDEX_CACHE


@functools.partial(
    pl.kernel,
    mesh=plsc.VectorSubcoreMesh(core_axis_name="c", subcore_axis_name="s"),
    out_type=jax.ShapeDtypeStruct((_SLICE, _D), jnp.float32),
    scratch_types=[
        pltpu.VMEM((_N_CHUNKS, _CHUNK), jnp.int32),
        pltpu.VMEM((_B_PER_W, _D), jnp.float32),
        pltpu.SemaphoreType.DMA((_N_CHUNKS,)),
        pltpu.SemaphoreType.DMA((_N_CHUNKS,)),
    ],
)
def _sc_gather(table_hbm, idx_hbm, out_hbm, idx_v, rows_v, gsem, wsem):
    wid = lax.axis_index("s") * _NC + lax.axis_index("c")
    base = wid * _B_PER_W
    pltpu.sync_copy(idx_hbm.at[wid], idx_v)
    gathers = [
        pltpu.async_copy(
            table_hbm.at[idx_v.at[j]],
            rows_v.at[pl.ds(j * _CHUNK, _CHUNK)],
            gsem.at[j],
        )
        for j in range(_N_CHUNKS)
    ]
    writes = []
    for j in range(_N_CHUNKS):
        gathers[j].wait()
        writes.append(
            pltpu.async_copy(
                rows_v.at[pl.ds(j * _CHUNK, _CHUNK)],
                out_hbm.at[pl.ds(base + j * _CHUNK, _CHUNK)],
                wsem.at[j],
            )
        )
    for w in writes:
        w.wait()


def kernel(x):
    index = jnp.asarray(_fixed_index())
    idx3 = index.reshape(_NW, _N_CHUNKS, _CHUNK)
    output = _sc_gather(x, idx3)
    return (output, index)


# speedup vs baseline: 1.0245x; 1.0245x over previous
import functools

import jax
import jax.numpy as jnp
import numpy as np
from jax import lax
from jax.experimental import pallas as pl
from jax.experimental.pallas import tpu as pltpu
from jax.experimental.pallas import tpu_sc as plsc

_N_ROWS = 100000
_SLICE = 16384
_D = 128

_NC = 2
_NS = 16
_NW = _NC * _NS
_B_PER_W = _SLICE // _NW
_CHUNK = 128
_N_CHUNKS = _B_PER_W // _CHUNK

_INDEX_CACHE = None


def _fixed_index() -> np.ndarray:
    global _INDEX_CACHE
    if _INDEX_CACHE is None:
        with jax.ensure_compile_time_eval():
            perm = jax.random.permutation(jax.random.key(1), _N_ROWS)
            _INDEX_CACHE = np.asarray(perm[:_SLICE])
    return _INDEX_CACHE


@functools.partial(
    pl.kernel,
    mesh=plsc.VectorSubcoreMesh(core_axis_name="c", subcore_axis_name="s"),
    out_type=(
        jax.ShapeDtypeStruct((_SLICE, _D), jnp.float32),
        jax.ShapeDtypeStruct((_SLICE,), jnp.int32),
    ),
    scratch_types=[
        pltpu.VMEM((_N_CHUNKS, _CHUNK), jnp.int32),
        pltpu.VMEM((_B_PER_W, _D), jnp.float32),
        pltpu.SemaphoreType.DMA,
        pltpu.SemaphoreType.DMA,
    ],
)
def _sc_gather(table_hbm, idx_hbm, out_hbm, idxout_hbm, idx_v, rows_v, gsem, isem):
    wid = lax.axis_index("s") * _NC + lax.axis_index("c")
    base = wid * _B_PER_W
    pltpu.sync_copy(idx_hbm.at[wid], idx_v)
    gathers = [
        pltpu.async_copy(
            table_hbm.at[idx_v.at[j]],
            rows_v.at[pl.ds(j * _CHUNK, _CHUNK)],
            gsem,
        )
        for j in range(_N_CHUNKS)
    ]
    iwrites = [
        pltpu.async_copy(
            idx_v.at[j],
            idxout_hbm.at[pl.ds(base + j * _CHUNK, _CHUNK)],
            isem,
        )
        for j in range(_N_CHUNKS)
    ]
    for c in gathers:
        c.wait()
    for c in iwrites:
        c.wait()
    pltpu.sync_copy(rows_v, out_hbm.at[pl.ds(base, _B_PER_W)])


def kernel(x):
    index = jnp.asarray(_fixed_index())
    idx3 = index.reshape(_NW, _N_CHUNKS, _CHUNK)
    output, index_out = _sc_gather(x, idx3)
    return (output, index_out)


# speedup vs baseline: 1.0316x; 1.0069x over previous
import functools

import jax
import jax.numpy as jnp
import numpy as np
from jax import lax
from jax.experimental import pallas as pl
from jax.experimental.pallas import tpu as pltpu
from jax.experimental.pallas import tpu_sc as plsc

_N_ROWS = 100000
_SLICE = 16384
_D = 128

_NC = 2
_NS = 16
_NW = _NC * _NS
_B_PER_W = _SLICE // _NW
_CHUNK = 128
_N_CHUNKS = _B_PER_W // _CHUNK

_INDEX_CACHE = None


def _fixed_index() -> np.ndarray:
    global _INDEX_CACHE
    if _INDEX_CACHE is None:
        with jax.ensure_compile_time_eval():
            perm = jax.random.permutation(jax.random.key(1), _N_ROWS)
            _INDEX_CACHE = np.asarray(perm[:_SLICE])
    return _INDEX_CACHE


@functools.partial(
    pl.kernel,
    mesh=plsc.VectorSubcoreMesh(core_axis_name="c", subcore_axis_name="s"),
    out_type=(
        jax.ShapeDtypeStruct((_SLICE, _D), jnp.float32),
        jax.ShapeDtypeStruct((_SLICE,), jnp.int32),
    ),
    scratch_types=[
        pltpu.VMEM((_B_PER_W,), jnp.int32),
        pltpu.VMEM((_B_PER_W, _D), jnp.float32),
        pltpu.SemaphoreType.DMA,
        pltpu.SemaphoreType.DMA,
    ],
)
def _sc_gather(table_hbm, idx_hbm, out_hbm, idxout_hbm, idx_v, rows_v, gsem, isem):
    wid = lax.axis_index("s") * _NC + lax.axis_index("c")
    base = wid * _B_PER_W
    pltpu.sync_copy(idx_hbm.at[pl.ds(base, _B_PER_W)], idx_v)
    gathers = [
        pltpu.async_copy(
            table_hbm.at[idx_v.at[pl.ds(j * _CHUNK, _CHUNK)]],
            rows_v.at[pl.ds(j * _CHUNK, _CHUNK)],
            gsem,
        )
        for j in range(_N_CHUNKS)
    ]
    iwrite = pltpu.async_copy(
        idx_v, idxout_hbm.at[pl.ds(base, _B_PER_W)], isem
    )
    for c in gathers:
        c.wait()
    iwrite.wait()
    pltpu.sync_copy(rows_v, out_hbm.at[pl.ds(base, _B_PER_W)])


def kernel(x):
    index = jnp.asarray(_fixed_index())
    output, index_out = _sc_gather(x, index)
    return (output, index_out)
